# Initial kernel scaffold; baseline (speedup 1.0000x reference)
import functools

import jax
import jax.numpy as jnp
from jax import lax
from jax.experimental import pallas as pl
from jax.experimental.pallas import tpu as pltpu
from jax.experimental.pallas import tpu_sc as plsc

WINDOW = 80


def _make_sc_scatter(N, E, D):
    info = plsc.get_sparse_core_info()
    NC, NS = info.num_cores, info.num_subcores
    NW = NC * NS
    assert E % NW == 0
    epw = E // NW
    assert epw % WINDOW == 0
    nwin = epw // WINDOW
    assert N % NS == 0
    rows_per = N // NS

    mesh = plsc.VectorSubcoreMesh(core_axis_name="c", subcore_axis_name="s")

    @functools.partial(
        pl.kernel,
        out_type=jax.ShapeDtypeStruct((NC, N, D), jnp.float32),
        mesh=mesh,
        scratch_types=[
            pltpu.VMEM((WINDOW,), jnp.int32),
            pltpu.VMEM((WINDOW,), jnp.int32),
            pltpu.VMEM((WINDOW, D), jnp.float32),
            pltpu.VMEM_SHARED((N, D), jnp.float32),
            pltpu.SemaphoreType.DMA,
        ],
    )
    def sc_scatter(src_hbm, dst_hbm, x_hbm, zeros_hbm, out_hbm,
                   sidx_v, didx_v, rows_v, acc_sh, sem):
        c = lax.axis_index("c")
        s = lax.axis_index("s")
        wid = s * NC + c
        base = wid * epw

        pltpu.sync_copy(zeros_hbm.at[pl.ds(s * rows_per, rows_per)],
                        acc_sh.at[pl.ds(s * rows_per, rows_per)])
        plsc.subcore_barrier()

        def body(i, _):
            off = base + i * WINDOW
            pltpu.sync_copy(src_hbm.at[pl.ds(off, WINDOW)], sidx_v)
            pltpu.async_copy(x_hbm.at[sidx_v], rows_v, sem).wait()
            pltpu.sync_copy(dst_hbm.at[pl.ds(off, WINDOW)], didx_v)
            pltpu.sync_copy(rows_v, acc_sh.at[didx_v], add=True)
            return 0

        lax.fori_loop(0, nwin, body, 0)
        plsc.subcore_barrier()

        pltpu.sync_copy(acc_sh.at[pl.ds(s * rows_per, rows_per)],
                        out_hbm.at[c, pl.ds(s * rows_per, rows_per)])

    return sc_scatter


def _tc_dense(p_ref, w_ref, b_ref, g_ref, bt_ref, o_ref):
    h = p_ref[0] + p_ref[1]
    h = lax.dot_general(h, w_ref[...], (((1,), (1,)), ((), ())),
                        preferred_element_type=jnp.float32)
    h = jnp.maximum(h + b_ref[...], 0.0)
    mean = jnp.mean(h, axis=0, keepdims=True)
    d = h - mean
    var = jnp.mean(d * d, axis=0, keepdims=True)
    o_ref[...] = d * lax.rsqrt(var + 1e-5) * g_ref[...] + bt_ref[...]


def kernel(x, edge_index, W, b, gamma, beta):
    N, D = x.shape
    E = edge_index.shape[1]
    src = edge_index[0]
    dst = edge_index[1]
    zeros = jnp.zeros((N, D), dtype=jnp.float32)

    partials = _make_sc_scatter(N, E, D)(src, dst, x, zeros)

    return pl.pallas_call(
        _tc_dense,
        out_shape=jax.ShapeDtypeStruct((N, D), jnp.float32),
    )(partials, W, b.reshape(1, D), gamma.reshape(1, D), beta.reshape(1, D))


# speedup vs baseline: 5.4899x; 5.4899x over previous
import functools

import jax
import jax.numpy as jnp
from jax import lax
from jax.experimental import pallas as pl
from jax.experimental.pallas import tpu as pltpu
from jax.experimental.pallas import tpu_sc as plsc

WINDOW = 80


def _make_sc_scatter(N, E, D):
    info = plsc.get_sparse_core_info()
    NC, NS = info.num_cores, info.num_subcores
    NW = NC * NS
    assert E % NW == 0
    epw = E // NW
    assert epw % WINDOW == 0
    nwin = epw // WINDOW
    rows_per = (N // NS) & ~7
    tail_off = NS * rows_per
    tail = N - tail_off

    mesh = plsc.VectorSubcoreMesh(core_axis_name="c", subcore_axis_name="s")

    @functools.partial(
        pl.kernel,
        out_type=jax.ShapeDtypeStruct((NC, N, D), jnp.float32),
        mesh=mesh,
        scratch_types=[
            pltpu.VMEM((WINDOW,), jnp.int32),
            pltpu.VMEM((WINDOW,), jnp.int32),
            pltpu.VMEM((WINDOW, D), jnp.float32),
            pltpu.VMEM_SHARED((N, D), jnp.float32),
            pltpu.SemaphoreType.DMA,
        ],
    )
    def sc_scatter(src_hbm, dst_hbm, x_hbm, zeros_hbm, out_hbm,
                   sidx_v, didx_v, rows_v, acc_sh, sem):
        c = lax.axis_index("c")
        s = lax.axis_index("s")
        wid = s * NC + c
        base = wid * epw

        pltpu.sync_copy(zeros_hbm.at[pl.ds(s * rows_per, rows_per)],
                        acc_sh.at[pl.ds(s * rows_per, rows_per)])
        if tail:
            @pl.when(s == NS - 1)
            def _():
                pltpu.sync_copy(zeros_hbm.at[pl.ds(tail_off, tail)],
                                acc_sh.at[pl.ds(tail_off, tail)])
        plsc.subcore_barrier()

        def body(i, _):
            off = base + i * WINDOW
            pltpu.sync_copy(src_hbm.at[pl.ds(off, WINDOW)], sidx_v)
            pltpu.async_copy(x_hbm.at[sidx_v], rows_v, sem).wait()
            pltpu.sync_copy(dst_hbm.at[pl.ds(off, WINDOW)], didx_v)
            pltpu.sync_copy(rows_v, acc_sh.at[didx_v], add=True)
            return 0

        lax.fori_loop(0, nwin, body, 0)
        plsc.subcore_barrier()

        pltpu.sync_copy(acc_sh.at[pl.ds(s * rows_per, rows_per)],
                        out_hbm.at[c, pl.ds(s * rows_per, rows_per)])
        if tail:
            @pl.when(s == NS - 1)
            def _():
                pltpu.sync_copy(acc_sh.at[pl.ds(tail_off, tail)],
                                out_hbm.at[c, pl.ds(tail_off, tail)])

    return sc_scatter


def _tc_dense(p_ref, w_ref, b_ref, g_ref, bt_ref, o_ref):
    h = p_ref[0] + p_ref[1]
    h = lax.dot_general(h, w_ref[...], (((1,), (1,)), ((), ())),
                        preferred_element_type=jnp.float32)
    h = jnp.maximum(h + b_ref[...], 0.0)
    mean = jnp.mean(h, axis=0, keepdims=True)
    d = h - mean
    var = jnp.mean(d * d, axis=0, keepdims=True)
    o_ref[...] = d * lax.rsqrt(var + 1e-5) * g_ref[...] + bt_ref[...]


def kernel(x, edge_index, W, b, gamma, beta):
    N, D = x.shape
    E = edge_index.shape[1]
    src = edge_index[0]
    dst = edge_index[1]
    zeros = jnp.zeros((N, D), dtype=jnp.float32)

    partials = _make_sc_scatter(N, E, D)(src, dst, x, zeros)

    return pl.pallas_call(
        _tc_dense,
        out_shape=jax.ShapeDtypeStruct((N, D), jnp.float32),
    )(partials, W, b.reshape(1, D), gamma.reshape(1, D), beta.reshape(1, D))


# speedup vs baseline: 10.7694x; 1.9617x over previous
import functools

import jax
import jax.numpy as jnp
from jax import lax
from jax.experimental import pallas as pl
from jax.experimental.pallas import tpu as pltpu
from jax.experimental.pallas import tpu_sc as plsc

WINDOW = 80


def _make_sc_scatter(N, E, D):
    info = plsc.get_sparse_core_info()
    NC, NS = info.num_cores, info.num_subcores
    NW = NC * NS
    assert E % NW == 0
    epw = E // NW
    assert epw % WINDOW == 0
    nwin = epw // WINDOW
    rows_per = (N // NS) & ~7
    tail_off = NS * rows_per
    tail = N - tail_off

    nwin_t = (nwin + 3) & ~3
    nwin_idx = nwin_t + 4
    nquads = nwin_t // 4
    mesh = plsc.VectorSubcoreMesh(core_axis_name="c", subcore_axis_name="s")

    @functools.partial(
        pl.kernel,
        out_type=jax.ShapeDtypeStruct((NC, N, D), jnp.float32),
        mesh=mesh,
        scratch_types=[
            pltpu.VMEM((4, 2, WINDOW), jnp.int32),
            pltpu.VMEM((WINDOW,), jnp.int32),
            pltpu.VMEM((WINDOW,), jnp.int32),
            pltpu.VMEM((WINDOW,), jnp.int32),
            pltpu.VMEM((WINDOW, D), jnp.float32),
            pltpu.VMEM((WINDOW, D), jnp.float32),
            pltpu.VMEM_SHARED((N + 8, D), jnp.float32),
            pltpu.SemaphoreType.DMA,
            pltpu.SemaphoreType.DMA,
            pltpu.SemaphoreType.DMA,
            pltpu.SemaphoreType.DMA,
            pltpu.SemaphoreType.DMA,
            pltpu.SemaphoreType.DMA,
        ],
    )
    def sc_scatter(idx_hbm, x_hbm, zeros_hbm, out_hbm,
                   idx_v, sidx_a, sidx_b, didx_v, rows_a, rows_b, acc_sh,
                   sem_a, sem_b, si0, si1, si2, si3):
        c = lax.axis_index("c")
        s = lax.axis_index("s")
        wid = s * NC + c
        rows = (rows_a, rows_b)
        sidx = (sidx_a, sidx_b)
        rsem = (sem_a, sem_b)
        isem = (si0, si1, si2, si3)

        def load_idx(w, q):
            pltpu.async_copy(idx_hbm.at[wid, w], idx_v.at[q], isem[q])

        def wait_idx(q):
            pltpu.make_async_copy(idx_hbm.at[wid, 0], idx_v.at[q],
                                  isem[q]).wait()

        def copy_idx(q, row, dst_1d):
            for k in range(WINDOW // 16):
                dst_1d[pl.ds(16 * k, 16)] = idx_v[q, row, pl.ds(16 * k, 16)]

        def start_gather(q, p):
            copy_idx(q, 0, sidx[p])
            pltpu.async_copy(x_hbm.at[sidx[p]], rows[p], rsem[p])

        def wait_gather(p):
            pltpu.make_async_copy(x_hbm.at[sidx[p]], rows[p], rsem[p]).wait()

        pltpu.sync_copy(zeros_hbm.at[pl.ds(s * rows_per, rows_per)],
                        acc_sh.at[pl.ds(s * rows_per, rows_per)])
        if tail:
            @pl.when(s == NS - 1)
            def _():
                pltpu.sync_copy(zeros_hbm.at[pl.ds(tail_off, tail)],
                                acc_sh.at[pl.ds(tail_off, tail)])
        plsc.subcore_barrier()

        for q in range(4):
            load_idx(q, q)
        wait_idx(0)
        start_gather(0, 0)
        wait_idx(1)
        start_gather(1, 1)

        def body(m, _):
            i0 = 4 * m
            for j in range(4):
                p = j % 2
                wait_gather(p)
                copy_idx(j, 1, didx_v)
                pltpu.sync_copy(rows[p], acc_sh.at[didx_v], add=True)
                load_idx(i0 + j + 4, j)
                wait_idx((j + 2) % 4)
                start_gather((j + 2) % 4, p)
            return 0

        lax.fori_loop(0, nquads, body, 0)
        wait_gather(0)
        wait_gather(1)
        wait_idx(2)
        wait_idx(3)
        plsc.subcore_barrier()

        pltpu.sync_copy(acc_sh.at[pl.ds(s * rows_per, rows_per)],
                        out_hbm.at[c, pl.ds(s * rows_per, rows_per)])
        if tail:
            @pl.when(s == NS - 1)
            def _():
                pltpu.sync_copy(acc_sh.at[pl.ds(tail_off, tail)],
                                out_hbm.at[c, pl.ds(tail_off, tail)])

    def wrapper(src, dst, x, zeros):
        npad = nwin_idx - nwin
        pad_src = (jnp.arange(npad * WINDOW, dtype=jnp.int32) * 1237) % N
        pad_src = jnp.broadcast_to(pad_src.reshape(1, npad, WINDOW),
                                   (NW, npad, WINDOW))
        pad_dst = jnp.full((NW, npad, WINDOW), N, dtype=jnp.int32)
        src_p = jnp.concatenate([src.reshape(NW, nwin, WINDOW), pad_src], 1)
        dst_p = jnp.concatenate([dst.reshape(NW, nwin, WINDOW), pad_dst], 1)
        idx4 = jnp.stack([src_p, dst_p], axis=2)
        return sc_scatter(idx4, x, zeros)

    return wrapper


def _tc_dense(p_ref, w_ref, b_ref, g_ref, bt_ref, o_ref):
    h = p_ref[0] + p_ref[1]
    h = lax.dot_general(h, w_ref[...], (((1,), (1,)), ((), ())),
                        preferred_element_type=jnp.float32)
    h = jnp.maximum(h + b_ref[...], 0.0)
    mean = jnp.mean(h, axis=0, keepdims=True)
    d = h - mean
    var = jnp.mean(d * d, axis=0, keepdims=True)
    o_ref[...] = d * lax.rsqrt(var + 1e-5) * g_ref[...] + bt_ref[...]


def kernel(x, edge_index, W, b, gamma, beta):
    N, D = x.shape
    E = edge_index.shape[1]
    src = edge_index[0]
    dst = edge_index[1]
    zeros = jnp.zeros((N, D), dtype=jnp.float32)

    partials = _make_sc_scatter(N, E, D)(src, dst, x, zeros)

    return pl.pallas_call(
        _tc_dense,
        out_shape=jax.ShapeDtypeStruct((N, D), jnp.float32),
    )(partials, W, b.reshape(1, D), gamma.reshape(1, D), beta.reshape(1, D))


# speedup vs baseline: 12.0024x; 1.1145x over previous
import functools

import jax
import jax.numpy as jnp
from jax import lax
from jax.experimental import pallas as pl
from jax.experimental.pallas import tpu as pltpu
from jax.experimental.pallas import tpu_sc as plsc

WINDOW = 80


def _make_sc_scatter(N, E, D):
    info = plsc.get_sparse_core_info()
    NC, NS = info.num_cores, info.num_subcores
    NW = NC * NS
    assert E % NW == 0
    epw = E // NW
    assert epw % WINDOW == 0
    nwin = epw // WINDOW
    rows_per = (N // NS) & ~7
    tail_off = NS * rows_per
    tail = N - tail_off

    nwin_t = 3 * ((nwin + 2) // 3)
    nwin_idx = nwin_t + 3
    ntriads = nwin_t // 3
    mesh = plsc.VectorSubcoreMesh(core_axis_name="c", subcore_axis_name="s")

    @functools.partial(
        pl.kernel,
        out_type=jax.ShapeDtypeStruct((NC, N, D), jnp.float32),
        mesh=mesh,
        scratch_types=[
            pltpu.VMEM((3, 2, WINDOW), jnp.int32),
            [pltpu.VMEM((WINDOW,), jnp.int32)] * 3,
            [pltpu.VMEM((WINDOW,), jnp.int32)] * 3,
            [pltpu.VMEM((WINDOW, D), jnp.float32)] * 3,
            pltpu.VMEM_SHARED((N + 8, D), jnp.float32),
            [pltpu.SemaphoreType.DMA] * 3,
            [pltpu.SemaphoreType.DMA] * 3,
            [pltpu.SemaphoreType.DMA] * 3,
        ],
    )
    def sc_scatter(idx_hbm, x_hbm, zeros_hbm, out_hbm,
                   idx_v, sidx, didx, rows, acc_sh, gsem, ssem, isem):
        c = lax.axis_index("c")
        s = lax.axis_index("s")
        wid = s * NC + c

        def load_idx(w, q):
            pltpu.async_copy(idx_hbm.at[wid, w], idx_v.at[q], isem[q])

        def wait_idx(q):
            pltpu.make_async_copy(idx_hbm.at[wid, 0], idx_v.at[q],
                                  isem[q]).wait()

        def copy_idx(q, row, dst_1d):
            for k in range(WINDOW // 16):
                dst_1d[pl.ds(16 * k, 16)] = idx_v[q, row, pl.ds(16 * k, 16)]

        def start_gather(q):
            copy_idx(q, 0, sidx[q])
            pltpu.async_copy(x_hbm.at[sidx[q]], rows[q], gsem[q])

        def wait_gather(q):
            pltpu.make_async_copy(x_hbm.at[sidx[q]], rows[q], gsem[q]).wait()

        def start_scatter(q):
            copy_idx(q, 1, didx[q])
            pltpu.async_copy(rows[q], acc_sh.at[didx[q]], ssem[q], add=True)

        def wait_scatter(q):
            pltpu.make_async_copy(rows[q], acc_sh.at[didx[q]], ssem[q]).wait()

        pltpu.sync_copy(zeros_hbm.at[pl.ds(s * rows_per, rows_per)],
                        acc_sh.at[pl.ds(s * rows_per, rows_per)])
        if tail:
            @pl.when(s == NS - 1)
            def _():
                pltpu.sync_copy(zeros_hbm.at[pl.ds(tail_off, tail)],
                                acc_sh.at[pl.ds(tail_off, tail)])
        plsc.subcore_barrier()

        for q in range(3):
            load_idx(q, q)
        wait_idx(0)
        start_gather(0)
        wait_idx(1)
        start_gather(1)

        def triad(i0, first):
            for j in range(3):
                jn = (j + 2) % 3
                wait_gather(j)
                start_scatter(j)
                load_idx(i0 + j + 3, j)
                wait_idx(jn)
                if not (first and j == 0):
                    wait_scatter(jn)
                start_gather(jn)

        triad(0, True)

        def body(m, _):
            triad(3 * m, False)
            return 0

        lax.fori_loop(1, ntriads, body, 0)
        wait_gather(0)
        wait_gather(1)
        wait_scatter(2)
        wait_idx(2)
        plsc.subcore_barrier()

        pltpu.sync_copy(acc_sh.at[pl.ds(s * rows_per, rows_per)],
                        out_hbm.at[c, pl.ds(s * rows_per, rows_per)])
        if tail:
            @pl.when(s == NS - 1)
            def _():
                pltpu.sync_copy(acc_sh.at[pl.ds(tail_off, tail)],
                                out_hbm.at[c, pl.ds(tail_off, tail)])

    def wrapper(src, dst, x, zeros):
        npad = nwin_idx - nwin
        pad_src = (jnp.arange(npad * WINDOW, dtype=jnp.int32) * 1237) % N
        pad_src = jnp.broadcast_to(pad_src.reshape(1, npad, WINDOW),
                                   (NW, npad, WINDOW))
        pad_dst = jnp.full((NW, npad, WINDOW), N, dtype=jnp.int32)
        src_p = jnp.concatenate([src.reshape(NW, nwin, WINDOW), pad_src], 1)
        dst_p = jnp.concatenate([dst.reshape(NW, nwin, WINDOW), pad_dst], 1)
        idx4 = jnp.stack([src_p, dst_p], axis=2)
        return sc_scatter(idx4, x, zeros)

    return wrapper


def _tc_dense(p_ref, w_ref, b_ref, g_ref, bt_ref, o_ref):
    h = p_ref[0] + p_ref[1]
    h = lax.dot_general(h, w_ref[...], (((1,), (1,)), ((), ())),
                        preferred_element_type=jnp.float32)
    h = jnp.maximum(h + b_ref[...], 0.0)
    mean = jnp.mean(h, axis=0, keepdims=True)
    d = h - mean
    var = jnp.mean(d * d, axis=0, keepdims=True)
    o_ref[...] = d * lax.rsqrt(var + 1e-5) * g_ref[...] + bt_ref[...]


def kernel(x, edge_index, W, b, gamma, beta):
    N, D = x.shape
    E = edge_index.shape[1]
    src = edge_index[0]
    dst = edge_index[1]
    zeros = jnp.zeros((N, D), dtype=jnp.float32)

    partials = _make_sc_scatter(N, E, D)(src, dst, x, zeros)

    return pl.pallas_call(
        _tc_dense,
        out_shape=jax.ShapeDtypeStruct((N, D), jnp.float32),
    )(partials, W, b.reshape(1, D), gamma.reshape(1, D), beta.reshape(1, D))


# speedup vs baseline: 12.2053x; 1.0169x over previous
import functools

import jax
import jax.numpy as jnp
from jax import lax
from jax.experimental import pallas as pl
from jax.experimental.pallas import tpu as pltpu
from jax.experimental.pallas import tpu_sc as plsc

WINDOW = 128


def _make_sc_scatter(N, E, D):
    info = plsc.get_sparse_core_info()
    NC, NS = info.num_cores, info.num_subcores
    NW = NC * NS
    assert E % WINDOW == 0
    nwin = E // WINDOW
    wq, wr = divmod(nwin, NW)
    nwt_u = wq + (1 if wr else 0)
    nwin_t = (nwt_u + 3) & ~3
    e_pad = (wq * (NW - 1) + wr + nwin_t + 4 - nwin) * WINDOW
    nquads = nwin_t // 4

    rows_per = (N // NS) & ~7
    tail_off = NS * rows_per
    tail = N - tail_off

    mesh = plsc.VectorSubcoreMesh(core_axis_name="c", subcore_axis_name="s")

    @functools.partial(
        pl.kernel,
        out_type=jax.ShapeDtypeStruct((NC, N, D), jnp.float32),
        mesh=mesh,
        scratch_types=[
            pltpu.VMEM((4, 2, WINDOW), jnp.int32),
            [pltpu.VMEM((WINDOW,), jnp.int32)] * 2,
            pltpu.VMEM((WINDOW,), jnp.int32),
            [pltpu.VMEM((WINDOW, D), jnp.float32)] * 2,
            pltpu.VMEM_SHARED((N + 8, D), jnp.float32),
            [pltpu.SemaphoreType.DMA] * 2,
            [pltpu.SemaphoreType.DMA] * 4,
        ],
    )
    def sc_scatter(edge_hbm, x_hbm, out_hbm,
                   idx_v, sidx, didx_v, rows, acc_sh, gsem, isem):
        c = lax.axis_index("c")
        s = lax.axis_index("s")
        wid = s * NC + c
        w0 = wq * wid + jnp.minimum(wid, wr)
        nwt = wq + jnp.where(wid < wr, 1, 0)

        def load_idx(w, q):
            off = pl.multiple_of((w0 + w) * WINDOW, WINDOW)
            pltpu.async_copy(edge_hbm.at[:, pl.ds(off, WINDOW)],
                             idx_v.at[q], isem[q])

        def wait_idx(q):
            pltpu.make_async_copy(edge_hbm.at[:, pl.ds(0, WINDOW)],
                                  idx_v.at[q], isem[q]).wait()

        def copy_src(q, p):
            for k in range(WINDOW // 16):
                sidx[p][pl.ds(16 * k, 16)] = idx_v[q, 0, pl.ds(16 * k, 16)]

        def copy_dst(q, real):
            garbage = jnp.full((16,), N, dtype=jnp.int32)
            for k in range(WINDOW // 16):
                v = idx_v[q, 1, pl.ds(16 * k, 16)]
                didx_v[pl.ds(16 * k, 16)] = jnp.where(real, v, garbage)

        def start_gather(q, p):
            copy_src(q, p)
            pltpu.async_copy(x_hbm.at[sidx[p]], rows[p], gsem[p])

        def wait_gather(p):
            pltpu.make_async_copy(x_hbm.at[sidx[p]], rows[p], gsem[p]).wait()

        zbuf = rows[0]

        def zrow(i, _):
            for k in range(D // 16):
                zbuf[i, pl.ds(16 * k, 16)] = jnp.zeros((16,), jnp.float32)
            return 0

        lax.fori_loop(0, WINDOW, zrow, 0)
        nfull, rem = divmod(rows_per, WINDOW)
        for t in range(nfull):
            pltpu.sync_copy(zbuf, acc_sh.at[pl.ds(s * rows_per + t * WINDOW,
                                                  WINDOW)])
        if rem:
            pltpu.sync_copy(zbuf.at[pl.ds(0, rem)],
                            acc_sh.at[pl.ds(s * rows_per + nfull * WINDOW,
                                            rem)])
        if tail:
            @pl.when(s == NS - 1)
            def _():
                pltpu.sync_copy(zbuf.at[pl.ds(0, tail)],
                                acc_sh.at[pl.ds(tail_off, tail)])
        plsc.subcore_barrier()

        for q in range(4):
            load_idx(q, q)
        wait_idx(0)
        start_gather(0, 0)
        wait_idx(1)
        start_gather(1, 1)

        def body(m, _):
            i0 = 4 * m
            for j in range(4):
                p = j % 2
                i = i0 + j
                wait_gather(p)
                copy_dst(j, i < nwt)
                pltpu.sync_copy(rows[p], acc_sh.at[didx_v], add=True)
                load_idx(i + 4, j)
                wait_idx((j + 2) % 4)
                start_gather((j + 2) % 4, p)
            return 0

        lax.fori_loop(0, nquads, body, 0)
        wait_gather(0)
        wait_gather(1)
        wait_idx(2)
        wait_idx(3)
        plsc.subcore_barrier()

        pltpu.sync_copy(acc_sh.at[pl.ds(s * rows_per, rows_per)],
                        out_hbm.at[c, pl.ds(s * rows_per, rows_per)])
        if tail:
            @pl.when(s == NS - 1)
            def _():
                pltpu.sync_copy(acc_sh.at[pl.ds(tail_off, tail)],
                                out_hbm.at[c, pl.ds(tail_off, tail)])

    def wrapper(edge_index, x):
        edge_pad = jnp.pad(edge_index, ((0, 0), (0, e_pad)))
        return sc_scatter(edge_pad, x)

    return wrapper


def _tc_dense(p_ref, w_ref, b_ref, g_ref, bt_ref, o_ref):
    h = p_ref[0] + p_ref[1]
    h = lax.dot_general(h, w_ref[...], (((1,), (1,)), ((), ())),
                        preferred_element_type=jnp.float32)
    h = jnp.maximum(h + b_ref[...], 0.0)
    mean = jnp.mean(h, axis=0, keepdims=True)
    d = h - mean
    var = jnp.mean(d * d, axis=0, keepdims=True)
    o_ref[...] = d * lax.rsqrt(var + 1e-5) * g_ref[...] + bt_ref[...]


def kernel(x, edge_index, W, b, gamma, beta):
    N, D = x.shape
    E = edge_index.shape[1]

    partials = _make_sc_scatter(N, E, D)(edge_index, x)

    return pl.pallas_call(
        _tc_dense,
        out_shape=jax.ShapeDtypeStruct((N, D), jnp.float32),
    )(partials, W, b.reshape(1, D), gamma.reshape(1, D), beta.reshape(1, D))
